# baseline (speedup 1.0000x reference)
import functools

import jax
import jax.numpy as jnp
from jax import lax
from jax.experimental import pallas as pl
from jax.experimental.pallas import tpu as pltpu
from jax.experimental.pallas import tpu_sc as plsc

_NUM_EMB = 1024
_DIM = 64
_ROWS = 16 * 32 * 32
_BLK = 2048
_COST = 0.25

_SC_CORES = 2
_SC_SUBCORES = 16
_SC_WORKERS = _SC_CORES * _SC_SUBCORES
_ROWS_PER_WORKER = _ROWS // _SC_WORKERS


def _vq_body(x_ref, wtn_ref, idx_ref, dsum_ref):
    x = x_ref[...]
    wtn = wtn_ref[...]
    xs = jnp.sum(x * x, axis=1, keepdims=True)
    ws = jnp.sum(wtn * wtn, axis=0, keepdims=True) * 0.25
    mmn = lax.dot_general(x, wtn, (((1,), (0,)), ((), ())),
                          preferred_element_type=jnp.float32)
    d = (xs + ws) + mmn
    dmin = jnp.min(d, axis=1, keepdims=True)
    cix = lax.broadcasted_iota(jnp.int32, d.shape, 1)
    idx = jnp.min(jnp.where(d == dmin, cix, _NUM_EMB), axis=1, keepdims=True)
    idx_ref[...] = idx

    @pl.when(pl.program_id(0) == 0)
    def _init():
        dsum_ref[...] = jnp.zeros_like(dsum_ref)

    dsum_ref[...] += jnp.sum(dmin).reshape(1, 1)


def _sc_gather_body(table_hbm, idx_hbm, out_hbm, idx_v, rows_v, sem):
    wid = lax.axis_index("s") * _SC_CORES + lax.axis_index("c")
    base = wid * _ROWS_PER_WORKER
    pltpu.sync_copy(idx_hbm.at[pl.ds(base, _ROWS_PER_WORKER)], idx_v)
    pltpu.async_copy(table_hbm.at[idx_v], rows_v, sem).wait()
    pltpu.sync_copy(rows_v, out_hbm.at[pl.ds(base, _ROWS_PER_WORKER)])


def kernel(inputs, W):
    x = jnp.transpose(inputs, (0, 2, 3, 1))
    xf = x.reshape(_ROWS, _DIM)
    wtn = -2.0 * W.T
    idx, dsum = pl.pallas_call(
        _vq_body,
        grid=(_ROWS // _BLK,),
        in_specs=[
            pl.BlockSpec((_BLK, _DIM), lambda i: (i, 0)),
            pl.BlockSpec((_DIM, _NUM_EMB), lambda i: (0, 0)),
        ],
        out_specs=[
            pl.BlockSpec((_BLK, 1), lambda i: (i, 0)),
            pl.BlockSpec((1, 1), lambda i: (0, 0)),
        ],
        out_shape=[
            jax.ShapeDtypeStruct((_ROWS, 1), jnp.int32),
            jax.ShapeDtypeStruct((1, 1), jnp.float32),
        ],
    )(xf, wtn)

    sc_gather = pl.kernel(
        _sc_gather_body,
        out_type=jax.ShapeDtypeStruct((_ROWS, _DIM), jnp.float32),
        mesh=plsc.VectorSubcoreMesh(core_axis_name="c", subcore_axis_name="s",
                                    num_cores=_SC_CORES,
                                    num_subcores=_SC_SUBCORES),
        scratch_types=[
            pltpu.VMEM((_ROWS_PER_WORKER,), jnp.int32),
            pltpu.VMEM((_ROWS_PER_WORKER, _DIM), jnp.float32),
            pltpu.SemaphoreType.DMA,
        ],
        compiler_params=pltpu.CompilerParams(use_tc_tiling_on_sc=False),
    )
    q = sc_gather(W, idx.reshape(_ROWS))

    loss = (1.0 + _COST) * dsum[0, 0] / (_ROWS * _DIM)
    qst = q.reshape(16, 32, 32, _DIM).transpose(0, 3, 1, 2)
    return qst, loss, idx


# speedup vs baseline: 1.0662x; 1.0662x over previous
import jax
import jax.numpy as jnp
from jax import lax
from jax.experimental import pallas as pl
from jax.experimental.pallas import tpu as pltpu
from jax.experimental.pallas import tpu_sc as plsc

_NUM_EMB = 1024
_DIM = 64
_N = 16
_PIX = 32 * 32
_ROWS = _N * _PIX
_COST = 0.25

_SC_CORES = 2
_SC_SUBCORES = 16
_SC_WORKERS = _SC_CORES * _SC_SUBCORES
_ROWS_PER_WORKER = _ROWS // _SC_WORKERS


def _vq_body(x_ref, wn_ref, w_ref, idx_ref, dsum_ref):
    x = x_ref[0]
    wn = wn_ref[...]
    xs = jnp.sum(x * x, axis=0, keepdims=True)
    ws = jnp.sum(w_ref[...] * w_ref[...], axis=1, keepdims=True)
    mmn = lax.dot_general(wn, x, (((1,), (0,)), ((), ())),
                          preferred_element_type=jnp.float32)
    d = (xs + ws) + mmn
    dmin = jnp.min(d, axis=0, keepdims=True)
    cix = lax.broadcasted_iota(jnp.int32, d.shape, 0)
    idx = jnp.min(jnp.where(d == dmin, cix, _NUM_EMB), axis=0, keepdims=True)
    idx_ref[0] = idx
    dsum_ref[...] = jnp.where(pl.program_id(0) == 0, 0.0, dsum_ref[...])
    dsum_ref[...] += jnp.sum(dmin).reshape(1, 1)


def _sc_gather_body(table_hbm, idx_hbm, out_hbm, idx_v, rows_v, sem):
    wid = lax.axis_index("s") * _SC_CORES + lax.axis_index("c")
    base = wid * _ROWS_PER_WORKER
    pltpu.sync_copy(idx_hbm.at[pl.ds(base, _ROWS_PER_WORKER)], idx_v)
    pltpu.async_copy(table_hbm.at[idx_v], rows_v, sem).wait()
    pltpu.sync_copy(rows_v, out_hbm.at[pl.ds(base, _ROWS_PER_WORKER)])


def kernel(inputs, W):
    xc = inputs.reshape(_N, _DIM, _PIX)
    wn = -2.0 * W
    idx3, dsum = pl.pallas_call(
        _vq_body,
        grid=(_N,),
        in_specs=[
            pl.BlockSpec((1, _DIM, _PIX), lambda i: (i, 0, 0)),
            pl.BlockSpec((_NUM_EMB, _DIM), lambda i: (0, 0)),
            pl.BlockSpec((_NUM_EMB, _DIM), lambda i: (0, 0)),
        ],
        out_specs=[
            pl.BlockSpec((1, 1, _PIX), lambda i: (i, 0, 0)),
            pl.BlockSpec((1, 1), lambda i: (0, 0)),
        ],
        out_shape=[
            jax.ShapeDtypeStruct((_N, 1, _PIX), jnp.int32),
            jax.ShapeDtypeStruct((1, 1), jnp.float32),
        ],
    )(xc, wn, W)

    sc_gather = pl.kernel(
        _sc_gather_body,
        out_type=jax.ShapeDtypeStruct((_ROWS, _DIM), jnp.float32),
        mesh=plsc.VectorSubcoreMesh(core_axis_name="c", subcore_axis_name="s",
                                    num_cores=_SC_CORES,
                                    num_subcores=_SC_SUBCORES),
        scratch_types=[
            pltpu.VMEM((_ROWS_PER_WORKER,), jnp.int32),
            pltpu.VMEM((_ROWS_PER_WORKER, _DIM), jnp.float32),
            pltpu.SemaphoreType.DMA,
        ],
        compiler_params=pltpu.CompilerParams(use_tc_tiling_on_sc=False),
    )
    q = sc_gather(W, idx3.reshape(_ROWS))

    loss = (1.0 + _COST) * dsum[0, 0] / (_ROWS * _DIM)
    qst = q.reshape(_N, 32, 32, _DIM).transpose(0, 3, 1, 2)
    return qst, loss, idx3.reshape(_ROWS, 1)


# speedup vs baseline: 1.4447x; 1.3550x over previous
import jax
import jax.numpy as jnp
from jax import lax
from jax.experimental import pallas as pl
from jax.experimental.pallas import tpu as pltpu
from jax.experimental.pallas import tpu_sc as plsc

_NUM_EMB = 1024
_DIM = 64
_N = 16
_PIX = 32 * 32
_ROWS = _N * _PIX
_COST = 0.25

_SC_CORES = 2
_SC_SUBCORES = 16
_SC_WORKERS = _SC_CORES * _SC_SUBCORES
_ROWS_PER_WORKER = _ROWS // _SC_WORKERS


def _vq_body(x_ref, wn_ref, w_ref, wt_ref, idx_ref, q_ref, dsum_ref):
    x = x_ref[0]
    wn = wn_ref[...]
    xs = jnp.sum(x * x, axis=0, keepdims=True)
    ws = jnp.sum(w_ref[...] * w_ref[...], axis=1, keepdims=True)
    mmn = lax.dot_general(wn, x, (((1,), (0,)), ((), ())),
                          preferred_element_type=jnp.float32)
    d = (xs + ws) + mmn
    dmin = jnp.min(d, axis=0, keepdims=True)
    cix = lax.broadcasted_iota(jnp.int32, d.shape, 0)
    idx = jnp.min(jnp.where(d == dmin, cix, _NUM_EMB), axis=0, keepdims=True)
    idx_ref[0] = idx
    enc = (cix == idx).astype(jnp.float32)
    q_ref[0] = lax.dot_general(wt_ref[...], enc, (((1,), (0,)), ((), ())),
                               preferred_element_type=jnp.float32)
    dsum_ref[...] = jnp.where(pl.program_id(0) == 0, 0.0, dsum_ref[...])
    dsum_ref[...] += jnp.sum(dmin).reshape(1, 1)


def _sc_gather_body(table_hbm, idx_hbm, out_hbm, idx_v, rows_v, sem):
    wid = lax.axis_index("s") * _SC_CORES + lax.axis_index("c")
    base = wid * _ROWS_PER_WORKER
    pltpu.sync_copy(idx_hbm.at[pl.ds(base, _ROWS_PER_WORKER)], idx_v)
    pltpu.async_copy(table_hbm.at[idx_v], rows_v, sem).wait()
    pltpu.sync_copy(rows_v, out_hbm.at[pl.ds(base, _ROWS_PER_WORKER)])


def kernel(inputs, W):
    xc = inputs.reshape(_N, _DIM, _PIX)
    wn = -2.0 * W
    wt = W.T
    idx3, q, dsum = pl.pallas_call(
        _vq_body,
        grid=(_N,),
        in_specs=[
            pl.BlockSpec((1, _DIM, _PIX), lambda i: (i, 0, 0)),
            pl.BlockSpec((_NUM_EMB, _DIM), lambda i: (0, 0)),
            pl.BlockSpec((_NUM_EMB, _DIM), lambda i: (0, 0)),
            pl.BlockSpec((_DIM, _NUM_EMB), lambda i: (0, 0)),
        ],
        out_specs=[
            pl.BlockSpec((1, 1, _PIX), lambda i: (i, 0, 0)),
            pl.BlockSpec((1, _DIM, _PIX), lambda i: (i, 0, 0)),
            pl.BlockSpec((1, 1), lambda i: (0, 0)),
        ],
        out_shape=[
            jax.ShapeDtypeStruct((_N, 1, _PIX), jnp.int32),
            jax.ShapeDtypeStruct((_N, _DIM, _PIX), jnp.float32),
            jax.ShapeDtypeStruct((1, 1), jnp.float32),
        ],
    )(xc, wn, W, wt)

    loss = (1.0 + _COST) * dsum[0, 0] / (_ROWS * _DIM)
    qst = q.reshape(_N, _DIM, 32, 32)
    return qst, loss, idx3.reshape(_ROWS, 1)
